# baseline (speedup 1.0000x reference)
import jax
import jax.numpy as jnp
import numpy as np
from jax.experimental import pallas as pl
from jax.experimental.pallas import tpu as pltpu

N = 2
A = 1
H = 100
W = 200
C = 80
L = 256
HWA = H * W * A
PRE_NMS_THRESH = 0.05
PRE_NMS_TOP_N = 1000
NMS_THRESH = 0.6
POST_TOP_N = 100
IMG_H, IMG_W = 800.0, 1333.0
_MAXR = float(np.log(1000.0 / 16.0))
_PAD = 1024


def _score_kernel(x_ref, c_ref, o_ref):
    x = x_ref[...][:, :C]
    s = jax.nn.sigmoid(x)
    cent = jax.nn.sigmoid(c_ref[...])
    w = s * cent
    o_ref[...] = jnp.where(s > PRE_NMS_THRESH, w, -1.0)


def _nms_kernel(a_ref, d_ref, tv_ref, lab_ref, bb_ref, vb_ref, keep_ref):
    a = a_ref[0]
    d = d_ref[0]
    ax1, ay1 = a[0:1, :], a[1:2, :]
    ax2, ay2 = a[2:3, :], a[3:4, :]
    dx, dy = d[0:1, :], d[1:2, :]
    dw, dh = d[2:3, :], d[3:4, :]
    wa = ax2 - ax1
    ha = ay2 - ay1
    cxa = (ax1 + ax2) * 0.5
    cya = (ay1 + ay2) * 0.5
    dw = jnp.clip(dw, -_MAXR, _MAXR)
    dh = jnp.clip(dh, -_MAXR, _MAXR)
    cx = cxa + dx * wa
    cy = cya + dy * ha
    bw_ = wa * jnp.exp(dw)
    bh_ = ha * jnp.exp(dh)
    x1 = jnp.clip(cx - bw_ * 0.5, 0.0, IMG_W)
    y1 = jnp.clip(cy - bh_ * 0.5, 0.0, IMG_H)
    x2 = jnp.clip(cx + bw_ * 0.5, 0.0, IMG_W)
    y2 = jnp.clip(cy + bh_ * 0.5, 0.0, IMG_H)
    bb_ref[0, 0:1, :] = x1
    bb_ref[0, 1:2, :] = y1
    bb_ref[0, 2:3, :] = x2
    bb_ref[0, 3:4, :] = y2

    tv = tv_ref[0]
    lab = lab_ref[0]
    bw = x2 - x1
    bh = y2 - y1
    vb = (tv > 0.0) & (bw > 0.0) & (bh > 0.0)
    vbf = vb.astype(jnp.float32)
    vb_ref[0] = vbf
    area = jnp.clip(bw, 0.0) * jnp.clip(bh, 0.0)
    idx = jax.lax.broadcasted_iota(jnp.int32, (1, _PAD), 1)

    def body(i, keep):
        m = (idx == i).astype(jnp.float32)
        xi1 = jnp.sum(x1 * m)
        yi1 = jnp.sum(y1 * m)
        xi2 = jnp.sum(x2 * m)
        yi2 = jnp.sum(y2 * m)
        ai = jnp.sum(area * m)
        li = jnp.sum(lab * m)
        gi = jnp.sum(keep * vbf * m)
        ix1 = jnp.maximum(x1, xi1)
        iy1 = jnp.maximum(y1, yi1)
        ix2 = jnp.minimum(x2, xi2)
        iy2 = jnp.minimum(y2, yi2)
        inter = jnp.clip(ix2 - ix1, 0.0) * jnp.clip(iy2 - iy1, 0.0)
        union = area + ai - inter
        iou = inter / jnp.maximum(union, 1e-9)
        sup = (iou > NMS_THRESH) & (idx > i) & vb & (lab == li) & (gi > 0.0)
        return jnp.where(sup, 0.0, keep)

    keep = jax.lax.fori_loop(0, PRE_NMS_TOP_N, body,
                             jnp.ones((1, _PAD), jnp.float32))
    keep_ref[0] = keep


def kernel(box_regression, centerness, anchors, box_cls, dot_product_logits):
    del box_cls
    R = N * HWA
    BR = 2000
    dpl2 = dot_product_logits.reshape(R, L)
    cent2 = centerness.reshape(R, 1)
    masked = pl.pallas_call(
        _score_kernel,
        grid=(R // BR,),
        in_specs=[
            pl.BlockSpec((BR, 128), lambda i: (i, 0)),
            pl.BlockSpec((BR, 1), lambda i: (i, 0)),
        ],
        out_specs=pl.BlockSpec((BR, C), lambda i: (i, 0)),
        out_shape=jax.ShapeDtypeStruct((R, C), jnp.float32),
    )(dpl2, cent2)

    K = 16
    M = HWA * C // K
    chunked = masked.reshape(N * K, M)
    c_vals, c_idx = jax.lax.top_k(chunked, PRE_NMS_TOP_N)
    base = (jnp.arange(N * K, dtype=jnp.int32) % K * M)[:, None]
    g_idx = (c_idx + base).reshape(N, K * PRE_NMS_TOP_N)
    g_vals = c_vals.reshape(N, K * PRE_NMS_TOP_N)
    top_vals, m_idx = jax.lax.top_k(g_vals, PRE_NMS_TOP_N)
    top_idx = jnp.take_along_axis(g_idx, m_idx, axis=-1)
    loc = top_idx // C
    labels = top_idx % C + 1

    box_reg = jnp.transpose(box_regression, (0, 2, 3, 1)).reshape(N, HWA, 4)
    deltas = jnp.take_along_axis(box_reg, loc[:, :, None], axis=1)
    anc = anchors[loc]

    pad = _PAD - PRE_NMS_TOP_N
    anc_t = jnp.pad(jnp.transpose(anc, (0, 2, 1)), ((0, 0), (0, 0), (0, pad)))
    del_t = jnp.pad(jnp.transpose(deltas, (0, 2, 1)), ((0, 0), (0, 0), (0, pad)))
    tv_p = jnp.pad(top_vals[:, None, :], ((0, 0), (0, 0), (0, pad)),
                   constant_values=-1.0)
    lab_p = jnp.pad(labels.astype(jnp.float32)[:, None, :],
                    ((0, 0), (0, 0), (0, pad)))

    bb_t, vbf, keepf = pl.pallas_call(
        _nms_kernel,
        grid=(N,),
        in_specs=[
            pl.BlockSpec((1, 4, _PAD), lambda n: (n, 0, 0)),
            pl.BlockSpec((1, 4, _PAD), lambda n: (n, 0, 0)),
            pl.BlockSpec((1, 1, _PAD), lambda n: (n, 0, 0)),
            pl.BlockSpec((1, 1, _PAD), lambda n: (n, 0, 0)),
        ],
        out_specs=[
            pl.BlockSpec((1, 4, _PAD), lambda n: (n, 0, 0)),
            pl.BlockSpec((1, 1, _PAD), lambda n: (n, 0, 0)),
            pl.BlockSpec((1, 1, _PAD), lambda n: (n, 0, 0)),
        ],
        out_shape=[
            jax.ShapeDtypeStruct((N, 4, _PAD), jnp.float32),
            jax.ShapeDtypeStruct((N, 1, _PAD), jnp.float32),
            jax.ShapeDtypeStruct((N, 1, _PAD), jnp.float32),
        ],
    )(anc_t, del_t, tv_p, lab_p)

    n = PRE_NMS_TOP_N
    bboxes = jnp.transpose(bb_t[:, :, :n], (0, 2, 1))
    valid = vbf[:, 0, :n] > 0.0
    keep = keepf[:, 0, :n] > 0.0

    scores = jnp.sqrt(jnp.maximum(top_vals, 1e-12))
    s = jnp.where(valid, scores, -1.0)
    order = jnp.argsort(-s, axis=-1)
    sb = jnp.take_along_axis(s, order, axis=-1)
    bb = jnp.take_along_axis(bboxes, order[:, :, None], axis=1)
    lb = jnp.take_along_axis(labels, order, axis=-1)
    kp = jnp.take_along_axis(keep & valid, order, axis=-1)
    final_s = jnp.where(kp, sb, -1.0)
    out_s, out_i = jax.lax.top_k(final_s, POST_TOP_N)
    out_b = jnp.take_along_axis(bb, out_i[:, :, None], axis=1)
    out_l = jnp.where(out_s > 0.0, jnp.take_along_axis(lb, out_i, axis=-1), 0)
    out_s = jnp.clip(out_s, 0.0)
    return out_b, out_s, out_l


# speedup vs baseline: 1.2801x; 1.2801x over previous
import jax
import jax.numpy as jnp
import numpy as np
from jax.experimental import pallas as pl
from jax.experimental.pallas import tpu as pltpu

N = 2
A = 1
H = 100
W = 200
C = 80
L = 256
HWA = H * W * A
PRE_NMS_THRESH = 0.05
PRE_NMS_TOP_N = 1000
NMS_THRESH = 0.6
POST_TOP_N = 100
IMG_H, IMG_W = 800.0, 1333.0
_MAXR = float(np.log(1000.0 / 16.0))
_PAD = 1024


def _score_kernel(x_ref, c_ref, o_ref):
    x = x_ref[...][:, :C]
    s = jax.nn.sigmoid(x)
    cent = jax.nn.sigmoid(c_ref[...])
    w = s * cent
    o_ref[...] = jnp.where(s > PRE_NMS_THRESH, w, -1.0)


def _nms_kernel(a_ref, d_ref, tv_ref, lab_ref, bb_ref, vb_ref, keep_ref):
    a = a_ref[0]
    d = d_ref[0]
    ax1, ay1 = a[0:1, :], a[1:2, :]
    ax2, ay2 = a[2:3, :], a[3:4, :]
    dx, dy = d[0:1, :], d[1:2, :]
    dw, dh = d[2:3, :], d[3:4, :]
    wa = ax2 - ax1
    ha = ay2 - ay1
    cxa = (ax1 + ax2) * 0.5
    cya = (ay1 + ay2) * 0.5
    dw = jnp.clip(dw, -_MAXR, _MAXR)
    dh = jnp.clip(dh, -_MAXR, _MAXR)
    cx = cxa + dx * wa
    cy = cya + dy * ha
    bw_ = wa * jnp.exp(dw)
    bh_ = ha * jnp.exp(dh)
    x1 = jnp.clip(cx - bw_ * 0.5, 0.0, IMG_W)
    y1 = jnp.clip(cy - bh_ * 0.5, 0.0, IMG_H)
    x2 = jnp.clip(cx + bw_ * 0.5, 0.0, IMG_W)
    y2 = jnp.clip(cy + bh_ * 0.5, 0.0, IMG_H)
    bb_ref[0, 0:1, :] = x1
    bb_ref[0, 1:2, :] = y1
    bb_ref[0, 2:3, :] = x2
    bb_ref[0, 3:4, :] = y2

    tv = tv_ref[0]
    lab = lab_ref[0]
    bw = x2 - x1
    bh = y2 - y1
    vb = (tv > 0.0) & (bw > 0.0) & (bh > 0.0)
    vbf = vb.astype(jnp.float32)
    vb_ref[0] = vbf
    area = jnp.clip(bw, 0.0) * jnp.clip(bh, 0.0)
    idx = jax.lax.broadcasted_iota(jnp.int32, (1, _PAD), 1)

    def body(i, keep):
        m = (idx == i).astype(jnp.float32)
        xi1 = jnp.sum(x1 * m)
        yi1 = jnp.sum(y1 * m)
        xi2 = jnp.sum(x2 * m)
        yi2 = jnp.sum(y2 * m)
        ai = jnp.sum(area * m)
        li = jnp.sum(lab * m)
        gi = jnp.sum(keep * vbf * m)
        ix1 = jnp.maximum(x1, xi1)
        iy1 = jnp.maximum(y1, yi1)
        ix2 = jnp.minimum(x2, xi2)
        iy2 = jnp.minimum(y2, yi2)
        inter = jnp.clip(ix2 - ix1, 0.0) * jnp.clip(iy2 - iy1, 0.0)
        union = area + ai - inter
        iou = inter / jnp.maximum(union, 1e-9)
        sup = (iou > NMS_THRESH) & (idx > i) & vb & (lab == li) & (gi > 0.0)
        return jnp.where(sup, 0.0, keep)

    keep = jax.lax.fori_loop(0, PRE_NMS_TOP_N, body,
                             jnp.ones((1, _PAD), jnp.float32))
    keep_ref[0] = keep


def kernel(box_regression, centerness, anchors, box_cls, dot_product_logits):
    del box_cls
    R = N * HWA
    BR = 2000
    dpl2 = dot_product_logits.reshape(R, L)
    cent2 = centerness.reshape(R, 1)
    masked = pl.pallas_call(
        _score_kernel,
        grid=(R // BR,),
        in_specs=[
            pl.BlockSpec((BR, 128), lambda i: (i, 0)),
            pl.BlockSpec((BR, 1), lambda i: (i, 0)),
        ],
        out_specs=pl.BlockSpec((BR, C), lambda i: (i, 0)),
        out_shape=jax.ShapeDtypeStruct((R, C), jnp.float32),
    )(dpl2, cent2)

    masked = masked.reshape(N, HWA * C)
    top_vals, top_idx = jax.lax.top_k(masked, PRE_NMS_TOP_N)
    loc = top_idx // C
    labels = top_idx % C + 1

    box_reg = jnp.transpose(box_regression, (0, 2, 3, 1)).reshape(N, HWA, 4)
    deltas = jnp.take_along_axis(box_reg, loc[:, :, None], axis=1)
    anc = anchors[loc]

    pad = _PAD - PRE_NMS_TOP_N
    anc_t = jnp.pad(jnp.transpose(anc, (0, 2, 1)), ((0, 0), (0, 0), (0, pad)))
    del_t = jnp.pad(jnp.transpose(deltas, (0, 2, 1)), ((0, 0), (0, 0), (0, pad)))
    tv_p = jnp.pad(top_vals[:, None, :], ((0, 0), (0, 0), (0, pad)),
                   constant_values=-1.0)
    lab_p = jnp.pad(labels.astype(jnp.float32)[:, None, :],
                    ((0, 0), (0, 0), (0, pad)))

    bb_t, vbf, keepf = pl.pallas_call(
        _nms_kernel,
        grid=(N,),
        in_specs=[
            pl.BlockSpec((1, 4, _PAD), lambda n: (n, 0, 0)),
            pl.BlockSpec((1, 4, _PAD), lambda n: (n, 0, 0)),
            pl.BlockSpec((1, 1, _PAD), lambda n: (n, 0, 0)),
            pl.BlockSpec((1, 1, _PAD), lambda n: (n, 0, 0)),
        ],
        out_specs=[
            pl.BlockSpec((1, 4, _PAD), lambda n: (n, 0, 0)),
            pl.BlockSpec((1, 1, _PAD), lambda n: (n, 0, 0)),
            pl.BlockSpec((1, 1, _PAD), lambda n: (n, 0, 0)),
        ],
        out_shape=[
            jax.ShapeDtypeStruct((N, 4, _PAD), jnp.float32),
            jax.ShapeDtypeStruct((N, 1, _PAD), jnp.float32),
            jax.ShapeDtypeStruct((N, 1, _PAD), jnp.float32),
        ],
    )(anc_t, del_t, tv_p, lab_p)

    n = PRE_NMS_TOP_N
    bboxes = jnp.transpose(bb_t[:, :, :n], (0, 2, 1))
    valid = vbf[:, 0, :n] > 0.0
    keep = keepf[:, 0, :n] > 0.0

    scores = jnp.sqrt(jnp.maximum(top_vals, 1e-12))
    s = jnp.where(valid, scores, -1.0)
    order = jnp.argsort(-s, axis=-1)
    sb = jnp.take_along_axis(s, order, axis=-1)
    bb = jnp.take_along_axis(bboxes, order[:, :, None], axis=1)
    lb = jnp.take_along_axis(labels, order, axis=-1)
    kp = jnp.take_along_axis(keep & valid, order, axis=-1)
    final_s = jnp.where(kp, sb, -1.0)
    out_s, out_i = jax.lax.top_k(final_s, POST_TOP_N)
    out_b = jnp.take_along_axis(bb, out_i[:, :, None], axis=1)
    out_l = jnp.where(out_s > 0.0, jnp.take_along_axis(lb, out_i, axis=-1), 0)
    out_s = jnp.clip(out_s, 0.0)
    return out_b, out_s, out_l
